# Initial kernel scaffold; baseline (speedup 1.0000x reference)
import jax
import jax.numpy as jnp
from jax.experimental import pallas as pl


def kernel(x, edge_index, batch, W1, b1, g1, be1, W2, b2, g2, be2, W3, b3, g3, be3, fcW, fcb):
    raise NotImplementedError("write your pallas kernel here")


# speedup vs baseline: 6.3624x; 6.3624x over previous
import jax
import jax.numpy as jnp
from jax import lax
from jax.experimental import pallas as pl
from jax.experimental.pallas import tpu as pltpu
from jax.experimental.pallas import tpu_sc as plsc

N_NODES = 10000
N_HEDGES = 10000
NNZ = 320000
FEAT = 128
NGROUP = 64

NC = 2
NS = 16
NW = NC * NS
EPW = NNZ // NW
CH = 80
NCHUNK = EPW // CH
NWB = 10
RWB = N_NODES // NWB

_f32 = jnp.float32


def _sc_stage_body(table, gidx, sidx, zeros, parts, idx_g, idx_s, rows, acc,
                   sem):
  c = lax.axis_index("c")
  s = lax.axis_index("s")
  w = c * NS + s

  @pl.when(s < NWB)
  def _():
    pltpu.sync_copy(zeros, acc.at[pl.ds(s * RWB, RWB)])

  plsc.subcore_barrier()

  def chunk(i, carry):
    off = w * EPW + i * CH
    pltpu.sync_copy(gidx.at[pl.ds(off, CH)], idx_g)
    pltpu.async_copy(table.at[idx_g], rows, sem).wait()
    pltpu.sync_copy(sidx.at[pl.ds(off, CH)], idx_s)
    pltpu.sync_copy(rows, acc.at[idx_s], add=True)
    return carry

  lax.fori_loop(0, NCHUNK, chunk, 0)
  plsc.subcore_barrier()

  @pl.when(s < NWB)
  def _():
    pltpu.sync_copy(acc.at[pl.ds(s * RWB, RWB)],
                    parts.at[pl.ds(c * N_NODES + s * RWB, RWB)])


def _sc_kernels():
  global _SC_CACHE
  if _SC_CACHE is None:
    mesh = plsc.VectorSubcoreMesh(core_axis_name="c", subcore_axis_name="s",
                                  num_cores=NC, num_subcores=NS)
    stage = pl.kernel(
        _sc_stage_body,
        out_type=jax.ShapeDtypeStruct((NC * N_NODES, FEAT), _f32),
        mesh=mesh,
        scratch_types=[
            pltpu.VMEM((CH,), jnp.int32),
            pltpu.VMEM((CH,), jnp.int32),
            pltpu.VMEM((CH, FEAT), _f32),
            pltpu.VMEM_SHARED((N_NODES, FEAT), _f32),
            pltpu.SemaphoreType.DMA,
        ],
    )
    count = pl.kernel(
        _sc_count_body,
        out_type=jax.ShapeDtypeStruct((NC * N_NODES, FEAT), _f32),
        mesh=mesh,
        scratch_types=[
            pltpu.VMEM((CH,), jnp.int32),
            pltpu.VMEM((CH, FEAT), _f32),
            pltpu.VMEM_SHARED((N_NODES, FEAT), _f32),
        ],
    )
    _SC_CACHE = (stage, count)
  return _SC_CACHE


_SC_CACHE = None


def _sc_count_body(sidx, ones_in, zeros, parts, idx_s, ones_v, acc):
  c = lax.axis_index("c")
  s = lax.axis_index("s")
  w = c * NS + s
  pltpu.sync_copy(ones_in, ones_v)

  @pl.when(s < NWB)
  def _():
    pltpu.sync_copy(zeros, acc.at[pl.ds(s * RWB, RWB)])

  plsc.subcore_barrier()

  def chunk(i, carry):
    off = w * EPW + i * CH
    pltpu.sync_copy(sidx.at[pl.ds(off, CH)], idx_s)
    pltpu.sync_copy(ones_v, acc.at[idx_s], add=True)
    return carry

  lax.fori_loop(0, NCHUNK, chunk, 0)
  plsc.subcore_barrier()

  @pl.when(s < NWB)
  def _():
    pltpu.sync_copy(acc.at[pl.ds(s * RWB, RWB)],
                    parts.at[pl.ds(c * N_NODES + s * RWB, RWB)])


def _inv_or_zero(colsum):
  return jnp.where(colsum > 0, 1.0 / jnp.where(colsum > 0, colsum, 1.0), 0.0)


def _mm_body(x_ref, w_ref, o_ref):
  o_ref[...] = jnp.dot(x_ref[...], w_ref[...], preferred_element_type=_f32)


_mm = pl.pallas_call(
    _mm_body,
    out_shape=jax.ShapeDtypeStruct((N_NODES, FEAT), _f32),
)


def _combine_edge_body(parts_ref, bparts_ref, o_ref):
  p = parts_ref[...]
  bp = bparts_ref[...]
  b = bp[:N_HEDGES, 0:1] + bp[N_HEDGES:, 0:1]
  o_ref[...] = (p[:N_HEDGES] + p[N_HEDGES:]) * _inv_or_zero(b)


_combine_edge = pl.pallas_call(
    _combine_edge_body,
    out_shape=jax.ShapeDtypeStruct((N_HEDGES, FEAT), _f32),
)


def _bn_relu(parts, dparts, bias, gamma, beta):
  d = dparts[:N_NODES, 0:1] + dparts[N_NODES:, 0:1]
  h = (parts[:N_NODES] + parts[N_NODES:]) * _inv_or_zero(d) + bias
  m = jnp.mean(h, axis=0, keepdims=True)
  v = jnp.mean((h - m) ** 2, axis=0, keepdims=True)
  return jnp.maximum((h - m) * lax.rsqrt(v + 1e-5) * gamma + beta, 0.0)


def _combine_node_mm_body(parts_ref, dparts_ref, b_ref, g_ref, be_ref, w_ref,
                          o_ref):
  y = _bn_relu(parts_ref[...], dparts_ref[...], b_ref[...], g_ref[...],
               be_ref[...])
  o_ref[...] = jnp.dot(y, w_ref[...], preferred_element_type=_f32)


_combine_node_mm = pl.pallas_call(
    _combine_node_mm_body,
    out_shape=jax.ShapeDtypeStruct((N_NODES, FEAT), _f32),
)


def _head_body(parts_ref, dparts_ref, b_ref, g_ref, be_ref, batch_ref,
               fcw_ref, fcb_ref, o_ref):
  y = _bn_relu(parts_ref[...], dparts_ref[...], b_ref[...], g_ref[...],
               be_ref[...])
  grp = batch_ref[...]
  onehot = (lax.broadcasted_iota(jnp.int32, (NGROUP, N_NODES), 0)
            == grp).astype(_f32)
  seg = jnp.dot(onehot, y, preferred_element_type=_f32)
  cnt = jnp.sum(onehot, axis=1, keepdims=True)
  pooled = seg / jnp.maximum(cnt, 1.0)
  logit = jnp.dot(pooled, fcw_ref[...], preferred_element_type=_f32)
  o_ref[...] = jax.nn.sigmoid(logit + fcb_ref[...])


_head = pl.pallas_call(
    _head_body,
    out_shape=jax.ShapeDtypeStruct((NGROUP, 1), _f32),
)


def kernel(x, edge_index, batch, W1, b1, g1, be1, W2, b2, g2, be2, W3, b3,
           g3, be3, fcW, fcb):
  ni = edge_index[0]
  ei = edge_index[1]
  zeros128 = jnp.zeros((RWB, FEAT), _f32)
  ones128 = jnp.ones((CH, FEAT), _f32)

  _sc_stage, _sc_count = _sc_kernels()
  dparts = _sc_count(ni, ones128, zeros128)
  bparts = _sc_count(ei, ones128, zeros128)

  xw = _mm(x, W1)
  parts = _sc_stage(xw, ni, ei, zeros128)
  e = _combine_edge(parts, bparts)
  parts = _sc_stage(e, ei, ni, zeros128)
  xw = _combine_node_mm(parts, dparts, b1.reshape(1, FEAT),
                        g1.reshape(1, FEAT), be1.reshape(1, FEAT), W2)

  parts = _sc_stage(xw, ni, ei, zeros128)
  e = _combine_edge(parts, bparts)
  parts = _sc_stage(e, ei, ni, zeros128)
  xw = _combine_node_mm(parts, dparts, b2.reshape(1, FEAT),
                        g2.reshape(1, FEAT), be2.reshape(1, FEAT), W3)

  parts = _sc_stage(xw, ni, ei, zeros128)
  e = _combine_edge(parts, bparts)
  parts = _sc_stage(e, ei, ni, zeros128)
  out = _head(parts, dparts, b3.reshape(1, FEAT), g3.reshape(1, FEAT),
              be3.reshape(1, FEAT), batch.reshape(1, N_NODES), fcW,
              fcb.reshape(1, 1))
  return out.reshape(-1)
